# baseline (speedup 1.0000x reference)
import functools

import jax
import jax.numpy as jnp
from jax import lax
from jax.experimental import pallas as pl
from jax.experimental.pallas import tpu as pltpu
from jax.experimental.pallas import tpu_sc as plsc

N_FINE = 100000
N_COARSE = 50000
IN_CH = 256
SKIP_CH = 128
OUT_CH = 256

NUM_CORES = 2
NUM_SUBCORES = 16
NUM_WORKERS = NUM_CORES * NUM_SUBCORES

CHUNK = 80
HALF = N_FINE // 2
N_CHUNKS_H = HALF // CHUNK
SLOTS_H = 20


def _sc_gather_body(idx_hbm, down_hbm, out_hbm, idx_all, rows0, rows1,
                    sem_g0, sem_g1, sem_w0, sem_w1):
    wid = lax.axis_index("s") * NUM_CORES + lax.axis_index("c")

    def gather(i, rows, sem):
        return pltpu.make_async_copy(down_hbm.at[idx_all.at[i]], rows, sem)

    def writeback(i, rows, sem):
        c = wid + i * NUM_WORKERS
        return pltpu.make_async_copy(rows, out_hbm.at[pl.ds(c * CHUNK, CHUNK)], sem)

    def valid(i):
        return wid + i * NUM_WORKERS < N_CHUNKS_H

    pltpu.sync_copy(idx_hbm.at[:, wid], idx_all)
    gather(0, rows0, sem_g0).start()

    def step(t, carry):
        i = 2 * t
        gather(i, rows0, sem_g0).wait()
        writeback(i, rows0, sem_w0).start()

        @pl.when(valid(i + 1))
        def _():
            @pl.when(t > 0)
            def _():
                writeback(i - 1, rows1, sem_w1).wait()

            gather(i + 1, rows1, sem_g1).start()

        @pl.when(valid(i + 1))
        def _():
            gather(i + 1, rows1, sem_g1).wait()
            writeback(i + 1, rows1, sem_w1).start()

        @pl.when(valid(i + 2))
        def _():
            writeback(i, rows0, sem_w0).wait()
            gather(i + 2, rows0, sem_g0).start()

        return carry

    lax.fori_loop(0, SLOTS_H // 2, step, 0)

    writeback(0, rows0, sem_w0).wait()
    writeback(0, rows1, sem_w1).wait()


_sc_gather_half = pl.kernel(
    _sc_gather_body,
    out_type=jax.ShapeDtypeStruct((HALF, IN_CH), jnp.float32),
    mesh=plsc.VectorSubcoreMesh(core_axis_name="c", subcore_axis_name="s"),
    scratch_types=[
        pltpu.VMEM((SLOTS_H, CHUNK), jnp.int32),
        pltpu.VMEM((CHUNK, IN_CH), jnp.float32),
        pltpu.VMEM((CHUNK, IN_CH), jnp.float32),
        pltpu.SemaphoreType.DMA,
        pltpu.SemaphoreType.DMA,
        pltpu.SemaphoreType.DMA,
        pltpu.SemaphoreType.DMA,
    ],
)


def _tc_fused_body(res_ref, gat_ref, wskip_ref, wproj_ref, bias_ref, out_ref):
    out_ref[...] = (
        jnp.dot(res_ref[...], wskip_ref[...], preferred_element_type=jnp.float32)
        + jnp.dot(gat_ref[...], wproj_ref[...], preferred_element_type=jnp.float32)
        + bias_ref[...]
    )


def _tc_fused_body2(res_ref, gat_ref, wskip_ref, wproj_ref, bias_ref, part_ref,
                    out_ref):
    del part_ref
    _tc_fused_body(res_ref, gat_ref, wskip_ref, wproj_ref, bias_ref, out_ref)


ROWS_BLK = 10000
GRID_H = HALF // ROWS_BLK

_W_SPECS = [
    pl.BlockSpec((SKIP_CH, OUT_CH), lambda i: (0, 0)),
    pl.BlockSpec((IN_CH, OUT_CH), lambda i: (0, 0)),
    pl.BlockSpec((1, OUT_CH), lambda i: (0, 0)),
]

_tc_first = pl.pallas_call(
    _tc_fused_body,
    grid=(GRID_H,),
    in_specs=[
        pl.BlockSpec((ROWS_BLK, SKIP_CH), lambda i: (i, 0)),
        pl.BlockSpec((ROWS_BLK, IN_CH), lambda i: (i, 0)),
        *_W_SPECS,
    ],
    out_specs=pl.BlockSpec((ROWS_BLK, OUT_CH), lambda i: (i, 0)),
    out_shape=jax.ShapeDtypeStruct((N_FINE, OUT_CH), jnp.float32),
)

_tc_second = pl.pallas_call(
    _tc_fused_body2,
    grid=(GRID_H,),
    in_specs=[
        pl.BlockSpec((ROWS_BLK, SKIP_CH), lambda i: (i + GRID_H, 0)),
        pl.BlockSpec((ROWS_BLK, IN_CH), lambda i: (i, 0)),
        *_W_SPECS,
        pl.BlockSpec(memory_space=pl.ANY),
    ],
    out_specs=pl.BlockSpec((ROWS_BLK, OUT_CH), lambda i: (i + GRID_H, 0)),
    out_shape=jax.ShapeDtypeStruct((N_FINE, OUT_CH), jnp.float32),
    input_output_aliases={5: 0},
)


def _half_idx(buffers_half):
    pad = SLOTS_H * NUM_WORKERS * CHUNK - HALF
    return jnp.pad(buffers_half, (0, pad)).reshape(SLOTS_H, NUM_WORKERS, CHUNK)


def kernel(residual, down, buffers, W_proj, b_proj, W_skip, b_skip):
    bias = (b_proj + b_skip).reshape(1, OUT_CH)
    g0 = _sc_gather_half(_half_idx(buffers[:HALF]), down)
    g1 = _sc_gather_half(_half_idx(buffers[HALF:]), down)
    part = _tc_first(residual, g0, W_skip, W_proj, bias)
    return _tc_second(residual, g1, W_skip, W_proj, bias, part)


# speedup vs baseline: 1.0050x; 1.0050x over previous
import functools

import jax
import jax.numpy as jnp
from jax import lax
from jax.experimental import pallas as pl
from jax.experimental.pallas import tpu as pltpu
from jax.experimental.pallas import tpu_sc as plsc

N_FINE = 100000
N_COARSE = 50000
IN_CH = 256
SKIP_CH = 128
OUT_CH = 256

NUM_CORES = 2
NUM_SUBCORES = 16
NUM_WORKERS = NUM_CORES * NUM_SUBCORES

CHUNK = 80
PARTS = (20000, 40000, 40000)


def _make_sc_gather(n_rows):
    n_chunks = n_rows // CHUNK
    slots = -(-n_chunks // NUM_WORKERS)
    slots += slots % 2

    def body(idx_hbm, down_hbm, out_hbm, idx_all, rows0, rows1,
             sem_g0, sem_g1, sem_w0, sem_w1):
        wid = lax.axis_index("s") * NUM_CORES + lax.axis_index("c")

        def gather(i, rows, sem):
            return pltpu.make_async_copy(down_hbm.at[idx_all.at[i]], rows, sem)

        def writeback(i, rows, sem):
            c = wid + i * NUM_WORKERS
            dst = out_hbm.at[pl.ds(c * CHUNK, CHUNK)]
            return pltpu.make_async_copy(rows, dst, sem)

        def valid(i):
            return wid + i * NUM_WORKERS < n_chunks

        pltpu.sync_copy(idx_hbm.at[:, wid], idx_all)
        gather(0, rows0, sem_g0).start()

        def step(t, carry):
            i = 2 * t
            gather(i, rows0, sem_g0).wait()
            writeback(i, rows0, sem_w0).start()

            @pl.when(valid(i + 1))
            def _():
                @pl.when(t > 0)
                def _():
                    writeback(i - 1, rows1, sem_w1).wait()

                gather(i + 1, rows1, sem_g1).start()

            @pl.when(valid(i + 1))
            def _():
                gather(i + 1, rows1, sem_g1).wait()
                writeback(i + 1, rows1, sem_w1).start()

            @pl.when(valid(i + 2))
            def _():
                writeback(i, rows0, sem_w0).wait()
                gather(i + 2, rows0, sem_g0).start()

            return carry

        lax.fori_loop(0, slots // 2, step, 0)

        writeback(0, rows0, sem_w0).wait()
        writeback(0, rows1, sem_w1).wait()

    gather_fn = pl.kernel(
        body,
        out_type=jax.ShapeDtypeStruct((n_rows, IN_CH), jnp.float32),
        mesh=plsc.VectorSubcoreMesh(core_axis_name="c", subcore_axis_name="s"),
        scratch_types=[
            pltpu.VMEM((slots, CHUNK), jnp.int32),
            pltpu.VMEM((CHUNK, IN_CH), jnp.float32),
            pltpu.VMEM((CHUNK, IN_CH), jnp.float32),
            pltpu.SemaphoreType.DMA,
            pltpu.SemaphoreType.DMA,
            pltpu.SemaphoreType.DMA,
            pltpu.SemaphoreType.DMA,
        ],
    )

    def run(buffers_part, down):
        pad = slots * NUM_WORKERS * CHUNK - n_rows
        idx = jnp.pad(buffers_part, (0, pad)).reshape(slots, NUM_WORKERS, CHUNK)
        return gather_fn(idx, down)

    return run


_sc_gathers = tuple(_make_sc_gather(n) for n in PARTS)


def _tc_fused_body(res_ref, gat_ref, wskip_ref, wproj_ref, bias_ref, out_ref):
    out_ref[...] = (
        jnp.dot(res_ref[...], wskip_ref[...], preferred_element_type=jnp.float32)
        + jnp.dot(gat_ref[...], wproj_ref[...], preferred_element_type=jnp.float32)
        + bias_ref[...]
    )


def _tc_fused_body2(res_ref, gat_ref, wskip_ref, wproj_ref, bias_ref, part_ref,
                    out_ref):
    del part_ref
    _tc_fused_body(res_ref, gat_ref, wskip_ref, wproj_ref, bias_ref, out_ref)


ROWS_BLK = 5000

_W_SPECS = [
    pl.BlockSpec((SKIP_CH, OUT_CH), lambda i: (0, 0)),
    pl.BlockSpec((IN_CH, OUT_CH), lambda i: (0, 0)),
    pl.BlockSpec((1, OUT_CH), lambda i: (0, 0)),
]


def _make_tc(n_rows, row_off, aliased):
    grid = n_rows // ROWS_BLK
    off = row_off // ROWS_BLK
    in_specs = [
        pl.BlockSpec((ROWS_BLK, SKIP_CH), lambda i, o=off: (i + o, 0)),
        pl.BlockSpec((ROWS_BLK, IN_CH), lambda i: (i, 0)),
        *_W_SPECS,
    ]
    if aliased:
        in_specs.append(pl.BlockSpec(memory_space=pl.ANY))
    return pl.pallas_call(
        _tc_fused_body2 if aliased else _tc_fused_body,
        grid=(grid,),
        in_specs=in_specs,
        out_specs=pl.BlockSpec((ROWS_BLK, OUT_CH), lambda i, o=off: (i + o, 0)),
        out_shape=jax.ShapeDtypeStruct((N_FINE, OUT_CH), jnp.float32),
        input_output_aliases={5: 0} if aliased else {},
    )


_ROW_OFFS = tuple(sum(PARTS[:k]) for k in range(len(PARTS)))
_tc_parts = tuple(
    _make_tc(n, o, k > 0) for k, (n, o) in enumerate(zip(PARTS, _ROW_OFFS))
)


def kernel(residual, down, buffers, W_proj, b_proj, W_skip, b_skip):
    bias = (b_proj + b_skip).reshape(1, OUT_CH)
    parts = [
        run(lax.dynamic_slice_in_dim(buffers, o, n), down)
        for run, n, o in zip(_sc_gathers, PARTS, _ROW_OFFS)
    ]
    out = _tc_parts[0](residual, parts[0], W_skip, W_proj, bias)
    for k in range(1, len(PARTS)):
        out = _tc_parts[k](residual, parts[k], W_skip, W_proj, bias, out)
    return out
